# baseline (speedup 1.0000x reference)
import functools

import jax
import jax.numpy as jnp
from jax import lax
from jax.experimental import pallas as pl
from jax.experimental.pallas import tpu as pltpu
from jax.experimental.pallas import tpu_sc as plsc

_B = 16384
_H = 512
_W = _H // 4
_NW = 32
_RPW = _B // _NW
_NG = _RPW // 16
_CH = _W // 4
_K = 0x01010101


def _srl(x, n):
    return lax.shift_right_logical(x, jnp.int32(n))


def _sc_positions(mask_words, u):
    mesh = plsc.VectorSubcoreMesh(core_axis_name="c", subcore_axis_name="s")

    @functools.partial(
        pl.kernel,
        mesh=mesh,
        out_type=jax.ShapeDtypeStruct((_B,), jnp.int32),
        scratch_types=[
            pltpu.VMEM((16 * _W,), jnp.int32),
            pltpu.VMEM((_CH * 16,), jnp.int32),
            pltpu.VMEM((_RPW,), jnp.float32),
            pltpu.VMEM((_RPW,), jnp.int32),
        ],
        compiler_params=pltpu.CompilerParams(needs_layout_passes=False),
    )
    def k(mask_hbm, u_hbm, out_hbm, buf, pref, u_v, out_v):
        wid = lax.axis_index("s") * 2 + lax.axis_index("c")
        row0 = wid * _RPW
        pltpu.sync_copy(u_hbm.at[pl.ds(row0, _RPW)], u_v)
        lanes = lax.iota(jnp.int32, 16)
        lbase = lanes * _W

        def group(g, carry):
            pltpu.sync_copy(mask_hbm.at[pl.ds((row0 + g * 16) * _W, 16 * _W)], buf)
            incl = jnp.zeros((16,), jnp.int32)
            for c in range(_CH):
                ws = [
                    plsc.load_gather(buf, [lbase + (4 * c + t)])
                    for t in range(4)
                ]
                s4 = ws[0] + ws[1] + ws[2] + ws[3]
                incl = incl + _srl(s4 * _K, 24)
                pref[pl.ds(c * 16, 16)] = incl
            total = incl
            uvec = u_v[pl.ds(g * 16, 16)]
            idx = (uvec * total.astype(jnp.float32)).astype(jnp.int32)
            idx = jnp.minimum(idx, jnp.maximum(total - 1, 0))
            pos = jnp.zeros((16,), jnp.int32)
            for s in (16, 8, 4, 2, 1):
                t = pos + s
                val = plsc.load_gather(pref, [(t - 1) * 16 + lanes])
                pos = jnp.where(val <= idx, t, pos)
            basev = plsc.load_gather(pref, [jnp.maximum(pos - 1, 0) * 16 + lanes])
            r16 = idx - jnp.where(pos > 0, basev, 0)
            w0 = plsc.load_gather(buf, [lbase + pos * 4])
            w1 = plsc.load_gather(buf, [lbase + pos * 4 + 1])
            w2 = plsc.load_gather(buf, [lbase + pos * 4 + 2])
            w3 = plsc.load_gather(buf, [lbase + pos * 4 + 3])
            s0 = _srl(w0 * _K, 24)
            s1 = s0 + _srl(w1 * _K, 24)
            s2 = s1 + _srl(w2 * _K, 24)
            b1 = s0 <= r16
            b2 = s1 <= r16
            b3 = s2 <= r16
            tw = b1.astype(jnp.int32) + b2.astype(jnp.int32) + b3.astype(jnp.int32)
            bw = jnp.where(b1, s0, 0)
            bw = jnp.where(b2, s1, bw)
            bw = jnp.where(b3, s2, bw)
            r4 = r16 - bw
            wv = jnp.where(b1, w1, w0)
            wv = jnp.where(b2, w2, wv)
            wv = jnp.where(b3, w3, wv)
            mm = wv * _K
            sb0 = mm & 0xFF
            sb1 = _srl(mm, 8) & 0xFF
            sb2 = _srl(mm, 16) & 0xFF
            tb = (
                (sb0 <= r4).astype(jnp.int32)
                + (sb1 <= r4).astype(jnp.int32)
                + (sb2 <= r4).astype(jnp.int32)
            )
            final = pos * 16 + tw * 4 + tb
            final = jnp.where(total > 0, final, 0)
            out_v[pl.ds(g * 16, 16)] = final
            return carry

        lax.fori_loop(0, _NG, group, 0)
        pltpu.sync_copy(out_v, out_hbm.at[pl.ds(row0, _RPW)])

    return k(mask_words.reshape(-1), u)


def _zeros_body(o_ref):
    o_ref[...] = jnp.zeros_like(o_ref)


def _tc_probs(B, H):
    blk = 2048
    return pl.pallas_call(
        _zeros_body,
        grid=(B // blk,),
        out_specs=pl.BlockSpec((blk, H), lambda i: (i, 0)),
        out_shape=jax.ShapeDtypeStruct((B, H), jnp.float32),
    )()


def kernel(context, forecast, forecast_mask):
    del context, forecast
    B, H = forecast_mask.shape
    u = jax.random.uniform(jax.random.key(42), (B,))
    mask_words = lax.bitcast_convert_type(
        forecast_mask.astype(jnp.uint8).reshape(B, H // 4, 4), jnp.int32
    )
    positions = _sc_positions(mask_words, u)
    probs = _tc_probs(B, H)
    return positions, probs


# speedup vs baseline: 2.2790x; 2.2790x over previous
import functools

import jax
import jax.numpy as jnp
from jax import lax
from jax.experimental import pallas as pl
from jax.experimental.pallas import tpu as pltpu
from jax.experimental.pallas import tpu_sc as plsc

_B = 16384
_H = 512
_W = _H // 4
_NW = 32
_RPW = _B // _NW
_NG = _RPW // 16
_CH = _W // 4
_K = 0x01010101


def _srl(x, n):
    return lax.shift_right_logical(x, jnp.int32(n))


def _sc_positions(words_flat, u):
    mesh = plsc.VectorSubcoreMesh(core_axis_name="c", subcore_axis_name="s")

    @functools.partial(
        pl.kernel,
        mesh=mesh,
        out_type=jax.ShapeDtypeStruct((_B,), jnp.int32),
        scratch_types=[
            pltpu.VMEM((_RPW * _W,), jnp.int32),
            pltpu.VMEM((_CH * 16,), jnp.int32),
            pltpu.VMEM((_RPW,), jnp.float32),
            pltpu.VMEM((_RPW,), jnp.int32),
        ],
        compiler_params=pltpu.CompilerParams(needs_layout_passes=False),
    )
    def k(words_hbm, u_hbm, out_hbm, buf, pref, u_v, out_v):
        wid = lax.axis_index("s") * 2 + lax.axis_index("c")
        row0 = wid * _RPW
        pltpu.sync_copy(u_hbm.at[pl.ds(row0, _RPW)], u_v)
        pltpu.sync_copy(words_hbm.at[pl.ds(row0 * _W, _RPW * _W)], buf)
        lanes = lax.iota(jnp.int32, 16)

        def group(g, carry):
            lbase = (g * 16 + lanes) * _W
            incl = jnp.zeros((16,), jnp.int32)
            for c in range(_CH):
                ws = [
                    plsc.load_gather(buf, [lbase + (4 * c + t)])
                    for t in range(4)
                ]
                s4 = ws[0] + ws[1] + ws[2] + ws[3]
                incl = incl + _srl(s4 * _K, 24)
                pref[pl.ds(c * 16, 16)] = incl
            total = incl
            uvec = u_v[pl.ds(g * 16, 16)]
            idx = (uvec * total.astype(jnp.float32)).astype(jnp.int32)
            idx = jnp.minimum(idx, jnp.maximum(total - 1, 0))
            pos = jnp.zeros((16,), jnp.int32)
            for s in (16, 8, 4, 2, 1):
                t = pos + s
                val = plsc.load_gather(pref, [(t - 1) * 16 + lanes])
                pos = jnp.where(val <= idx, t, pos)
            basev = plsc.load_gather(pref, [jnp.maximum(pos - 1, 0) * 16 + lanes])
            r16 = idx - jnp.where(pos > 0, basev, 0)
            w0 = plsc.load_gather(buf, [lbase + pos * 4])
            w1 = plsc.load_gather(buf, [lbase + pos * 4 + 1])
            w2 = plsc.load_gather(buf, [lbase + pos * 4 + 2])
            w3 = plsc.load_gather(buf, [lbase + pos * 4 + 3])
            s0 = _srl(w0 * _K, 24)
            s1 = s0 + _srl(w1 * _K, 24)
            s2 = s1 + _srl(w2 * _K, 24)
            b1 = s0 <= r16
            b2 = s1 <= r16
            b3 = s2 <= r16
            tw = b1.astype(jnp.int32) + b2.astype(jnp.int32) + b3.astype(jnp.int32)
            bw = jnp.where(b1, s0, 0)
            bw = jnp.where(b2, s1, bw)
            bw = jnp.where(b3, s2, bw)
            r4 = r16 - bw
            wv = jnp.where(b1, w1, w0)
            wv = jnp.where(b2, w2, wv)
            wv = jnp.where(b3, w3, wv)
            mm = wv * _K
            sb0 = mm & 0xFF
            sb1 = _srl(mm, 8) & 0xFF
            sb2 = _srl(mm, 16) & 0xFF
            tb = (
                (sb0 <= r4).astype(jnp.int32)
                + (sb1 <= r4).astype(jnp.int32)
                + (sb2 <= r4).astype(jnp.int32)
            )
            final = pos * 16 + tw * 4 + tb
            final = jnp.where(total > 0, final, 0)
            out_v[pl.ds(g * 16, 16)] = final
            return carry

        lax.fori_loop(0, _NG, group, 0)
        pltpu.sync_copy(out_v, out_hbm.at[pl.ds(row0, _RPW)])

    return k(words_flat, u)


def _words_body(mask_ref, words_ref):
    m = mask_ref[...].astype(jnp.bfloat16)
    j = lax.broadcasted_iota(jnp.int32, (_H, _W), 0)
    w = lax.broadcasted_iota(jnp.int32, (_H, _W), 1)
    grp = (j // 4) == w
    r = j % 4
    wl = jnp.where(r == 0, 1.0, jnp.where(r == 1, 256.0, 0.0))
    wh = jnp.where(r == 2, 1.0, jnp.where(r == 3, 256.0, 0.0))
    p_lo = jnp.where(grp, wl, 0.0).astype(jnp.bfloat16)
    p_hi = jnp.where(grp, wh, 0.0).astype(jnp.bfloat16)
    lo = jnp.dot(m, p_lo, preferred_element_type=jnp.float32).astype(jnp.int32)
    hi = jnp.dot(m, p_hi, preferred_element_type=jnp.float32).astype(jnp.int32)
    words_ref[...] = lo + hi * 65536


def _tc_words(mask):
    blk = 2048
    return pl.pallas_call(
        _words_body,
        grid=(_B // blk,),
        in_specs=[pl.BlockSpec((blk, _H), lambda i: (i, 0))],
        out_specs=pl.BlockSpec((blk, _W), lambda i: (i, 0)),
        out_shape=jax.ShapeDtypeStruct((_B, _W), jnp.int32),
    )(mask)


def _zeros_body(o_ref):
    o_ref[...] = jnp.zeros_like(o_ref)


def _tc_probs(B, H):
    blk = 2048
    return pl.pallas_call(
        _zeros_body,
        grid=(B // blk,),
        out_specs=pl.BlockSpec((blk, H), lambda i: (i, 0)),
        out_shape=jax.ShapeDtypeStruct((B, H), jnp.float32),
    )()


def kernel(context, forecast, forecast_mask):
    del context, forecast
    B, H = forecast_mask.shape
    u = jax.random.uniform(jax.random.key(42), (B,))
    words = _tc_words(forecast_mask.astype(jnp.int8))
    positions = _sc_positions(words.reshape(-1), u)
    probs = _tc_probs(B, H)
    return positions, probs


# speedup vs baseline: 3.4390x; 1.5090x over previous
import functools

import jax
import jax.numpy as jnp
from jax import lax
from jax.experimental import pallas as pl
from jax.experimental.pallas import tpu as pltpu
from jax.experimental.pallas import tpu_sc as plsc

_B = 16384
_H = 512
_W = _H // 4
_NW = 32
_RPW = _B // _NW
_NG = _RPW // 16
_K = 0x01010101


def _srl(x, n):
    return lax.shift_right_logical(x, jnp.int32(n))


def _sc_positions(words_flat, pref_flat, u):
    mesh = plsc.VectorSubcoreMesh(core_axis_name="c", subcore_axis_name="s")

    @functools.partial(
        pl.kernel,
        mesh=mesh,
        out_type=jax.ShapeDtypeStruct((_B,), jnp.int32),
        scratch_types=[
            pltpu.VMEM((_RPW * _W,), jnp.int32),
            pltpu.VMEM((_RPW,), jnp.float32),
            pltpu.VMEM((_RPW,), jnp.int32),
            pltpu.VMEM((_RPW,), jnp.int32),
            pltpu.VMEM((_RPW,), jnp.int32),
            pltpu.VMEM((_RPW,), jnp.int32),
            pltpu.SemaphoreType.DMA,
        ],
        compiler_params=pltpu.CompilerParams(needs_layout_passes=False),
    )
    def k(words_hbm, pref_hbm, u_hbm, out_hbm,
          pref_v, u_v, widx_v, r4_v, wv_v, out_v, sem):
        wid = lax.axis_index("s") * 2 + lax.axis_index("c")
        row0 = wid * _RPW
        pltpu.sync_copy(u_hbm.at[pl.ds(row0, _RPW)], u_v)
        pltpu.sync_copy(pref_hbm.at[pl.ds(row0 * _W, _RPW * _W)], pref_v)
        lanes = lax.iota(jnp.int32, 16)

        def search(g, carry):
            lb = (g * 16 + lanes) * _W
            total = plsc.load_gather(pref_v, [lb + (_W - 1)])
            uvec = u_v[pl.ds(g * 16, 16)]
            idx = (uvec * total.astype(jnp.float32)).astype(jnp.int32)
            idx = jnp.minimum(idx, jnp.maximum(total - 1, 0))
            pos = jnp.zeros((16,), jnp.int32)
            for s in (64, 32, 16, 8, 4, 2, 1):
                t = pos + s
                val = plsc.load_gather(pref_v, [lb + t - 1])
                pos = jnp.where(val <= idx, t, pos)
            basev = plsc.load_gather(pref_v, [lb + jnp.maximum(pos - 1, 0)])
            base = jnp.where(pos > 0, basev, 0)
            empty = total <= 0
            pos = jnp.where(empty, 0, pos)
            widx_v[pl.ds(g * 16, 16)] = (row0 + g * 16 + lanes) * _W + pos
            r4_v[pl.ds(g * 16, 16)] = jnp.where(empty, -1, idx - base)
            return carry

        lax.fori_loop(0, _NG, search, 0)
        pltpu.async_copy(words_hbm.at[widx_v], wv_v, sem).wait()

        def resolve(g, carry):
            wv = wv_v[pl.ds(g * 16, 16)]
            wq = widx_v[pl.ds(g * 16, 16)]
            r4 = r4_v[pl.ds(g * 16, 16)]
            mm = wv * _K
            sb0 = mm & 0xFF
            sb1 = _srl(mm, 8) & 0xFF
            sb2 = _srl(mm, 16) & 0xFF
            tb = (
                (sb0 <= r4).astype(jnp.int32)
                + (sb1 <= r4).astype(jnp.int32)
                + (sb2 <= r4).astype(jnp.int32)
            )
            out_v[pl.ds(g * 16, 16)] = (wq & (_W - 1)) * 4 + tb
            return carry

        lax.fori_loop(0, _NG, resolve, 0)
        pltpu.sync_copy(out_v, out_hbm.at[pl.ds(row0, _RPW)])

    return k(words_flat, pref_flat, u)


def _pack_body(mask_ref, words_ref, pref_ref):
    m = mask_ref[...].astype(jnp.bfloat16)
    j = lax.broadcasted_iota(jnp.int32, (_H, _W), 0)
    w = lax.broadcasted_iota(jnp.int32, (_H, _W), 1)
    grp = (j // 4) == w
    r = j % 4
    wl = jnp.where(r == 0, 1.0, jnp.where(r == 1, 256.0, 0.0))
    wh = jnp.where(r == 2, 1.0, jnp.where(r == 3, 256.0, 0.0))
    p_lo = jnp.where(grp, wl, 0.0).astype(jnp.bfloat16)
    p_hi = jnp.where(grp, wh, 0.0).astype(jnp.bfloat16)
    p_pref = jnp.where(j // 4 <= w, 1.0, 0.0).astype(jnp.bfloat16)
    lo = jnp.dot(m, p_lo, preferred_element_type=jnp.float32).astype(jnp.int32)
    hi = jnp.dot(m, p_hi, preferred_element_type=jnp.float32).astype(jnp.int32)
    words_ref[...] = lo + hi * 65536
    pref_ref[...] = jnp.dot(
        m, p_pref, preferred_element_type=jnp.float32
    ).astype(jnp.int32)


def _tc_pack(mask):
    blk = 2048
    return pl.pallas_call(
        _pack_body,
        grid=(_B // blk,),
        in_specs=[pl.BlockSpec((blk, _H), lambda i: (i, 0))],
        out_specs=[
            pl.BlockSpec((blk, _W), lambda i: (i, 0)),
            pl.BlockSpec((blk, _W), lambda i: (i, 0)),
        ],
        out_shape=[
            jax.ShapeDtypeStruct((_B, _W), jnp.int32),
            jax.ShapeDtypeStruct((_B, _W), jnp.int32),
        ],
    )(mask)


def _zeros_body(o_ref):
    o_ref[...] = jnp.zeros_like(o_ref)


def _tc_probs(B, H):
    blk = 2048
    return pl.pallas_call(
        _zeros_body,
        grid=(B // blk,),
        out_specs=pl.BlockSpec((blk, H), lambda i: (i, 0)),
        out_shape=jax.ShapeDtypeStruct((B, H), jnp.float32),
    )()


def kernel(context, forecast, forecast_mask):
    del context, forecast
    B, H = forecast_mask.shape
    u = jax.random.uniform(jax.random.key(42), (B,))
    words, pref = _tc_pack(forecast_mask.astype(jnp.int8))
    positions = _sc_positions(words.reshape(-1), pref.reshape(-1), u)
    probs = _tc_probs(B, H)
    return positions, probs


# speedup vs baseline: 3.9772x; 1.1565x over previous
import functools

import numpy as np

import jax
import jax.numpy as jnp
from jax import lax
from jax.experimental import pallas as pl
from jax.experimental.pallas import tpu as pltpu
from jax.experimental.pallas import tpu_sc as plsc

_B = 16384
_H = 512
_W = _H // 4
_NW = 32
_RPW = _B // _NW
_NG = _RPW // 16


def _srl(x, n):
    return lax.shift_right_logical(x, jnp.int32(n))


def _pack_matrix():
    jj = np.arange(_H)
    ww = np.arange(_W)
    wgt = np.array([73.0, 72.0, 64.0, 512.0])[jj % 4]
    p = np.where(
        (jj[:, None] // 4) < ww[None, :],
        1024.0,
        np.where((jj[:, None] // 4) == ww[None, :], wgt[:, None], 0.0),
    )
    return jnp.asarray(p, dtype=jnp.bfloat16)


def _sc_positions(comb_flat, u):
    mesh = plsc.VectorSubcoreMesh(core_axis_name="c", subcore_axis_name="s")

    @functools.partial(
        pl.kernel,
        mesh=mesh,
        out_type=jax.ShapeDtypeStruct((_B,), jnp.int32),
        scratch_types=[
            pltpu.VMEM((_RPW * _W,), jnp.int32),
            pltpu.VMEM((_RPW,), jnp.float32),
            pltpu.VMEM((_RPW,), jnp.int32),
            pltpu.VMEM((_RPW,), jnp.int32),
            pltpu.VMEM((_RPW,), jnp.int32),
            pltpu.VMEM((_RPW,), jnp.int32),
            pltpu.SemaphoreType.DMA,
        ],
        compiler_params=pltpu.CompilerParams(needs_layout_passes=False),
    )
    def k(comb_hbm, u_hbm, out_hbm,
          comb_v, u_v, widx_v, r4_v, wv_v, out_v, sem):
        wid = lax.axis_index("s") * 2 + lax.axis_index("c")
        row0 = wid * _RPW
        pltpu.sync_copy(u_hbm.at[pl.ds(row0, _RPW)], u_v)
        pltpu.sync_copy(comb_hbm.at[pl.ds(row0 * _W, _RPW * _W)], comb_v)
        lanes = lax.iota(jnp.int32, 16)

        def search(g, carry):
            lb = (g * 16 + lanes) * _W
            last = plsc.load_gather(comb_v, [lb + (_W - 1)])
            total = _srl(last, 10) + (_srl(last, 6) & 7) + (_srl(last, 9) & 1)
            uvec = u_v[pl.ds(g * 16, 16)]
            idx = (uvec * total.astype(jnp.float32)).astype(jnp.int32)
            idx = jnp.minimum(idx, jnp.maximum(total - 1, 0))
            pos = jnp.zeros((16,), jnp.int32)
            best = jnp.zeros((16,), jnp.int32)
            for s in (64, 32, 16, 8, 4, 2, 1):
                t = pos + s
                val = _srl(plsc.load_gather(comb_v, [lb + t]), 10)
                take = val <= idx
                pos = jnp.where(take, t, pos)
                best = jnp.where(take, val, best)
            empty = total <= 0
            pos = jnp.where(empty, 0, pos)
            widx_v[pl.ds(g * 16, 16)] = (row0 + g * 16 + lanes) * _W + pos
            r4_v[pl.ds(g * 16, 16)] = jnp.where(empty, -1, idx - best)
            return carry

        lax.fori_loop(0, _NG, search, 0)
        pltpu.async_copy(comb_hbm.at[widx_v], wv_v, sem).wait()

        def resolve(g, carry):
            info = wv_v[pl.ds(g * 16, 16)] & 1023
            wq = widx_v[pl.ds(g * 16, 16)]
            r4 = r4_v[pl.ds(g * 16, 16)]
            s0 = info & 7
            s1 = _srl(info, 3) & 7
            s2 = _srl(info, 6) & 7
            tb = (
                (s0 <= r4).astype(jnp.int32)
                + (s1 <= r4).astype(jnp.int32)
                + (s2 <= r4).astype(jnp.int32)
            )
            out_v[pl.ds(g * 16, 16)] = (wq & (_W - 1)) * 4 + tb
            return carry

        lax.fori_loop(0, _NG, resolve, 0)
        pltpu.sync_copy(out_v, out_hbm.at[pl.ds(row0, _RPW)])

    return k(comb_flat, u)


def _pack_body(mask_ref, p_ref, comb_ref):
    m = mask_ref[...].astype(jnp.bfloat16)
    comb_ref[...] = jnp.dot(
        m, p_ref[...], preferred_element_type=jnp.float32
    ).astype(jnp.int32)


def _tc_pack(mask, p):
    blk = 4096
    return pl.pallas_call(
        _pack_body,
        grid=(_B // blk,),
        in_specs=[
            pl.BlockSpec((blk, _H), lambda i: (i, 0)),
            pl.BlockSpec((_H, _W), lambda i: (0, 0)),
        ],
        out_specs=pl.BlockSpec((blk, _W), lambda i: (i, 0)),
        out_shape=jax.ShapeDtypeStruct((_B, _W), jnp.int32),
    )(mask, p)


def _zeros_body(o_ref):
    o_ref[...] = jnp.zeros_like(o_ref)


def _tc_probs(B, H):
    blk = 2048
    return pl.pallas_call(
        _zeros_body,
        grid=(B // blk,),
        out_specs=pl.BlockSpec((blk, H), lambda i: (i, 0)),
        out_shape=jax.ShapeDtypeStruct((B, H), jnp.float32),
    )()


def kernel(context, forecast, forecast_mask):
    del context, forecast
    B, H = forecast_mask.shape
    u = jax.random.uniform(jax.random.key(42), (B,))
    comb = _tc_pack(forecast_mask.view(jnp.int8), _pack_matrix())
    positions = _sc_positions(comb.reshape(-1), u)
    probs = _tc_probs(B, H)
    return positions, probs
